# baseline (device time: 430359 ns/iter reference)
import jax
import jax.numpy as jnp
from jax import lax
from jax.experimental import pallas as pl
from jax.experimental.pallas import tpu as pltpu

N_DEV = 16
SQ = 512
D = 1024
HEADS_PER = 8
DH = 128
SKV = 2048
SCALE = 0.08838834764831843
CH = SQ // N_DEV
N_STEPS = 2 * (N_DEV - 1)
N_BLK = 4
import os
_COMPUTE_ONLY_PROBE = os.environ.get("KERNEL_COMPUTE_ONLY") == "1"


def _body(x_ref, wq_ref, wo_ref, k_ref, v_ref, out_ref,
          q_ref, attn_ref, acc_ref, rs_sbuf, rs_rbuf, ag_src, ag_rbuf,
          send_sems, recv_sems, exit_sem):
    my = lax.axis_index("i")

    barrier = pltpu.get_barrier_semaphore()
    for d in range(1, N_DEV):
        pl.semaphore_signal(barrier, inc=1,
                            device_id=(lax.rem(my + d, N_DEV),),
                            device_id_type=pl.DeviceIdType.MESH)
    pl.semaphore_wait(barrier, N_DEV - 1)

    q_ref[...] = lax.dot_general(
        x_ref[...], wq_ref[...], (((1,), (0,)), ((), ())),
        preferred_element_type=jnp.float32)

    rs = [
        pltpu.make_async_remote_copy(
            src_ref=rs_sbuf.at[d - 1],
            dst_ref=rs_rbuf.at[d - 1],
            send_sem=send_sems.at[d - 1],
            recv_sem=recv_sems.at[d - 1],
            device_id=(lax.rem(my + d, N_DEV),),
            device_id_type=pl.DeviceIdType.MESH,
        )
        for d in range(1, N_DEV)
    ]

    RB = SQ // N_BLK
    for b in range(N_BLK):
        r0 = b * RB
        for h in range(HEADS_PER):
            q_h = q_ref[r0:r0 + RB, h * DH:(h + 1) * DH]
            s = lax.dot_general(q_h, k_ref[:, h * DH:(h + 1) * DH],
                                (((1,), (1,)), ((), ())),
                                preferred_element_type=jnp.float32) * SCALE
            m = jnp.max(s, axis=1, keepdims=True)
            p = jnp.exp(s - m)
            l = jnp.sum(p, axis=1, keepdims=True)
            o = lax.dot_general(p, v_ref[:, h * DH:(h + 1) * DH],
                                (((1,), (0,)), ((), ())),
                                preferred_element_type=jnp.float32)
            attn_ref[r0:r0 + RB, h * DH:(h + 1) * DH] = o / l

        acc_ref[4 * b:4 * (b + 1)] = lax.dot_general(
            attn_ref[r0:r0 + RB, :], wo_ref[...], (((1,), (0,)), ((), ())),
            preferred_element_type=jnp.float32).reshape(4, CH, D)

        if not _COMPUTE_ONLY_PROBE:
            for d in range(1, N_DEV):
                tgt = lax.rem(my + d, N_DEV)

                @pl.when(tgt // 4 == b)
                def _(d=d, tgt=tgt):
                    rs_sbuf[d - 1] = acc_ref[tgt]
                    rs[d - 1].start()

    if not _COMPUTE_ONLY_PROBE:
        total = acc_ref[my]
        for s in range(N_DEV - 1):
            rs[s].wait_recv()
            total = total + rs_rbuf[s]
        ag_src[...] = total
        acc_ref[my] = total

        ag = []
        for d in range(1, N_DEV):
            r = pltpu.make_async_remote_copy(
                src_ref=ag_src,
                dst_ref=ag_rbuf.at[d - 1],
                send_sem=send_sems.at[N_DEV - 1 + d - 1],
                recv_sem=recv_sems.at[N_DEV - 1 + d - 1],
                device_id=(lax.rem(my + d, N_DEV),),
                device_id_type=pl.DeviceIdType.MESH,
            )
            r.start()
            ag.append(r)
        for s in range(N_DEV - 1):
            ag[s].wait_recv()
            acc_ref[lax.rem(my - s - 1 + N_DEV, N_DEV)] = ag_rbuf[s]

        for r in rs:
            r.wait_send()
        for r in ag:
            r.wait_send()

    out_ref[...] = acc_ref[...].reshape(SQ, D)

    for d in range(1, N_DEV):
        pl.semaphore_signal(exit_sem, inc=1,
                            device_id=(lax.rem(my + d, N_DEV),),
                            device_id_type=pl.DeviceIdType.MESH)
    pl.semaphore_wait(exit_sem, N_DEV - 1)


def kernel(x, Wq, Wo, K_ext, V_ext):
    my = lax.axis_index("i")
    Kh = lax.dynamic_index_in_dim(
        jnp.reshape(K_ext[0], (SKV, N_DEV, HEADS_PER * DH)), my, axis=1,
        keepdims=False)
    Vh = lax.dynamic_index_in_dim(
        jnp.reshape(V_ext[0], (SKV, N_DEV, HEADS_PER * DH)), my, axis=1,
        keepdims=False)

    out = pl.pallas_call(
        _body,
        out_shape=jax.ShapeDtypeStruct((SQ, D), jnp.float32),
        in_specs=[pl.BlockSpec(memory_space=pltpu.VMEM)] * 5,
        out_specs=pl.BlockSpec(memory_space=pltpu.VMEM),
        scratch_shapes=[
            pltpu.VMEM((SQ, D), jnp.float32),
            pltpu.VMEM((SQ, D), jnp.float32),
            pltpu.VMEM((N_DEV, CH, D), jnp.float32),
            pltpu.VMEM((N_DEV - 1, CH, D), jnp.float32),
            pltpu.VMEM((N_DEV - 1, CH, D), jnp.float32),
            pltpu.VMEM((CH, D), jnp.float32),
            pltpu.VMEM((N_DEV - 1, CH, D), jnp.float32),
            pltpu.SemaphoreType.DMA((N_STEPS,)),
            pltpu.SemaphoreType.DMA((N_STEPS,)),
            pltpu.SemaphoreType.REGULAR,
        ],
        compiler_params=pltpu.CompilerParams(
            collective_id=0, vmem_limit_bytes=64 * 1024 * 1024),
    )(x[0], Wq, Wo, Kh, Vh)
    return out[None]


# device time: 112940 ns/iter; 3.8105x vs baseline; 3.8105x over previous
import jax
import jax.numpy as jnp
from jax import lax
from jax.experimental import pallas as pl
from jax.experimental.pallas import tpu as pltpu

N_DEV = 16
SQ = 512
D = 1024
HEADS_PER = 8
DH = 128
SKV = 2048
SCALE = 0.08838834764831843
CH = SQ // N_DEV
N_STEPS = 2 * (N_DEV - 1)
N_BLK = 4
import os
_COMPUTE_ONLY_PROBE = os.environ.get("KERNEL_COMPUTE_ONLY") == "1"


def _body(x_ref, wq_ref, wo_ref, k_ref, v_ref, out_ref,
          q_ref, attn_ref, acc_ref, rs_sbuf, rs_rbuf, ag_src, ag_rbuf,
          send_sems, recv_sems, exit_sem):
    my = lax.axis_index("i")

    barrier = pltpu.get_barrier_semaphore()
    for d in range(1, N_DEV):
        pl.semaphore_signal(barrier, inc=1,
                            device_id=(lax.rem(my + d, N_DEV),),
                            device_id_type=pl.DeviceIdType.MESH)
    pl.semaphore_wait(barrier, N_DEV - 1)

    q_ref[...] = lax.dot_general(
        x_ref[...], wq_ref[...], (((1,), (0,)), ((), ())),
        preferred_element_type=jnp.float32)

    rs = [
        pltpu.make_async_remote_copy(
            src_ref=rs_sbuf.at[d - 1],
            dst_ref=rs_rbuf.at[d - 1],
            send_sem=send_sems.at[d - 1],
            recv_sem=recv_sems.at[d - 1],
            device_id=(lax.rem(my + d, N_DEV),),
            device_id_type=pl.DeviceIdType.MESH,
        )
        for d in range(1, N_DEV)
    ]

    RB = SQ // N_BLK
    for b in range(N_BLK):
        r0 = b * RB
        for h in range(HEADS_PER):
            q_h = q_ref[r0:r0 + RB, h * DH:(h + 1) * DH]
            s = lax.dot_general(q_h, k_ref[h], (((1,), (1,)), ((), ())),
                                preferred_element_type=jnp.float32) * SCALE
            m = jnp.max(s, axis=1, keepdims=True)
            p = jnp.exp(s - m)
            l = jnp.sum(p, axis=1, keepdims=True)
            o = lax.dot_general(p, v_ref[h], (((1,), (0,)), ((), ())),
                                preferred_element_type=jnp.float32)
            attn_ref[r0:r0 + RB, h * DH:(h + 1) * DH] = o / l

        acc_ref[4 * b:4 * (b + 1)] = lax.dot_general(
            attn_ref[r0:r0 + RB, :], wo_ref[...], (((1,), (0,)), ((), ())),
            preferred_element_type=jnp.float32).reshape(4, CH, D)

        if not _COMPUTE_ONLY_PROBE:
            for d in range(1, N_DEV):
                tgt = lax.rem(my + d, N_DEV)

                @pl.when(tgt // 4 == b)
                def _(d=d, tgt=tgt):
                    rs_sbuf[d - 1] = acc_ref[tgt]
                    rs[d - 1].start()

    if not _COMPUTE_ONLY_PROBE:
        total = acc_ref[my]
        for s in range(N_DEV - 1):
            rs[s].wait_recv()
            total = total + rs_rbuf[s]
        ag_src[...] = total
        acc_ref[my] = total

        ag = []
        for d in range(1, N_DEV):
            r = pltpu.make_async_remote_copy(
                src_ref=ag_src,
                dst_ref=ag_rbuf.at[d - 1],
                send_sem=send_sems.at[N_DEV - 1 + d - 1],
                recv_sem=recv_sems.at[N_DEV - 1 + d - 1],
                device_id=(lax.rem(my + d, N_DEV),),
                device_id_type=pl.DeviceIdType.MESH,
            )
            r.start()
            ag.append(r)
        for s in range(N_DEV - 1):
            ag[s].wait_recv()
            acc_ref[lax.rem(my - s - 1 + N_DEV, N_DEV)] = ag_rbuf[s]

        for r in rs:
            r.wait_send()
        for r in ag:
            r.wait_send()

    out_ref[...] = acc_ref[...].reshape(SQ, D)

    for d in range(1, N_DEV):
        pl.semaphore_signal(exit_sem, inc=1,
                            device_id=(lax.rem(my + d, N_DEV),),
                            device_id_type=pl.DeviceIdType.MESH)
    pl.semaphore_wait(exit_sem, N_DEV - 1)


def kernel(x, Wq, Wo, K_ext, V_ext):
    my = lax.axis_index("i")
    Kh = jnp.transpose(
        lax.dynamic_slice_in_dim(K_ext[0], my * HEADS_PER, HEADS_PER, axis=1),
        (1, 0, 2))
    Vh = jnp.transpose(
        lax.dynamic_slice_in_dim(V_ext[0], my * HEADS_PER, HEADS_PER, axis=1),
        (1, 0, 2))

    out = pl.pallas_call(
        _body,
        out_shape=jax.ShapeDtypeStruct((SQ, D), jnp.float32),
        in_specs=[pl.BlockSpec(memory_space=pltpu.VMEM)] * 5,
        out_specs=pl.BlockSpec(memory_space=pltpu.VMEM),
        scratch_shapes=[
            pltpu.VMEM((SQ, D), jnp.float32),
            pltpu.VMEM((SQ, D), jnp.float32),
            pltpu.VMEM((N_DEV, CH, D), jnp.float32),
            pltpu.VMEM((N_DEV - 1, CH, D), jnp.float32),
            pltpu.VMEM((N_DEV - 1, CH, D), jnp.float32),
            pltpu.VMEM((CH, D), jnp.float32),
            pltpu.VMEM((N_DEV - 1, CH, D), jnp.float32),
            pltpu.SemaphoreType.DMA((N_STEPS,)),
            pltpu.SemaphoreType.DMA((N_STEPS,)),
            pltpu.SemaphoreType.REGULAR,
        ],
        compiler_params=pltpu.CompilerParams(
            collective_id=0, vmem_limit_bytes=64 * 1024 * 1024),
    )(x[0], Wq, Wo, Kh, Vh)
    return out[None]


# device time: 111643 ns/iter; 3.8548x vs baseline; 1.0116x over previous
import jax
import jax.numpy as jnp
from jax import lax
from jax.experimental import pallas as pl
from jax.experimental.pallas import tpu as pltpu

N_DEV = 16
SQ = 512
D = 1024
HEADS_PER = 8
DH = 128
SKV = 2048
SCALE = 0.08838834764831843
CH = SQ // N_DEV
N_STEPS = 2 * (N_DEV - 1)
N_BLK = 4
import os
_COMPUTE_ONLY_PROBE = os.environ.get("KERNEL_COMPUTE_ONLY") == "1"


def _body(x_ref, wq_ref, wo_ref, k_ref, v_ref, out_ref,
          q_ref, attn_ref, acc_ref, rs_sbuf, rs_rbuf, bf_ref,
          send_sems, recv_sems, exit_sem):
    my = lax.axis_index("i")

    barrier = pltpu.get_barrier_semaphore()
    for d in range(1, N_DEV):
        pl.semaphore_signal(barrier, inc=1,
                            device_id=(lax.rem(my + d, N_DEV),),
                            device_id_type=pl.DeviceIdType.MESH)
    pl.semaphore_wait(barrier, N_DEV - 1)

    q_ref[...] = lax.dot_general(
        x_ref[...], wq_ref[...], (((1,), (0,)), ((), ())),
        preferred_element_type=jnp.float32)

    rs = [
        pltpu.make_async_remote_copy(
            src_ref=rs_sbuf.at[d - 1],
            dst_ref=rs_rbuf.at[d - 1],
            send_sem=send_sems.at[d - 1],
            recv_sem=recv_sems.at[d - 1],
            device_id=(lax.rem(my + d, N_DEV),),
            device_id_type=pl.DeviceIdType.MESH,
        )
        for d in range(1, N_DEV)
    ]

    RB = SQ // N_BLK
    for b in range(N_BLK):
        r0 = b * RB
        for h in range(HEADS_PER):
            q_h = q_ref[r0:r0 + RB, h * DH:(h + 1) * DH]
            s = lax.dot_general(q_h, k_ref[h], (((1,), (1,)), ((), ())),
                                preferred_element_type=jnp.float32) * SCALE
            m = jnp.max(s, axis=1, keepdims=True)
            p = jnp.exp(s - m)
            l = jnp.sum(p, axis=1, keepdims=True)
            o = lax.dot_general(p, v_ref[h], (((1,), (0,)), ((), ())),
                                preferred_element_type=jnp.float32)
            attn_ref[r0:r0 + RB, h * DH:(h + 1) * DH] = o / l

        acc_ref[4 * b:4 * (b + 1)] = lax.dot_general(
            attn_ref[r0:r0 + RB, :], wo_ref[...], (((1,), (0,)), ((), ())),
            preferred_element_type=jnp.float32).reshape(4, CH, D)

        if not _COMPUTE_ONLY_PROBE:
            for d in range(1, N_DEV):
                tgt = lax.rem(my + d, N_DEV)

                @pl.when(tgt // 4 == b)
                def _(d=d, tgt=tgt):
                    rs_sbuf[d - 1] = acc_ref[tgt]
                    rs[d - 1].start()

    if not _COMPUTE_ONLY_PROBE:
        total = acc_ref[my]
        for s in range(N_DEV - 1):
            rs[s].wait_recv()
            total = total + rs_rbuf[s]
        acc_ref[my] = total

        bf_ref[0] = total
        descs = {}
        prev = [0]
        pending = []
        for m in (8, 2, 4, 1):
            for i in pending:
                descs[i].wait_recv()
            pending = []
            for j in prev:
                i = j + m
                r = pltpu.make_async_remote_copy(
                    src_ref=bf_ref.at[j],
                    dst_ref=bf_ref.at[i],
                    send_sem=send_sems.at[N_DEV - 2 + i],
                    recv_sem=recv_sems.at[N_DEV - 2 + i],
                    device_id=(jnp.bitwise_xor(my, m),),
                    device_id_type=pl.DeviceIdType.MESH,
                )
                r.start()
                descs[i] = r
                pending.append(i)
            prev = prev + pending
        for i in pending:
            descs[i].wait_recv()
        for j in range(1, N_DEV):
            acc_ref[jnp.bitwise_xor(my, j)] = bf_ref[j]

        for r in rs:
            r.wait_send()
        for r in descs.values():
            r.wait_send()

    out_ref[...] = acc_ref[...].reshape(SQ, D)

    for d in range(1, N_DEV):
        pl.semaphore_signal(exit_sem, inc=1,
                            device_id=(lax.rem(my + d, N_DEV),),
                            device_id_type=pl.DeviceIdType.MESH)
    pl.semaphore_wait(exit_sem, N_DEV - 1)


def kernel(x, Wq, Wo, K_ext, V_ext):
    my = lax.axis_index("i")
    Kh = jnp.transpose(
        lax.dynamic_slice_in_dim(K_ext[0], my * HEADS_PER, HEADS_PER, axis=1),
        (1, 0, 2))
    Vh = jnp.transpose(
        lax.dynamic_slice_in_dim(V_ext[0], my * HEADS_PER, HEADS_PER, axis=1),
        (1, 0, 2))

    out = pl.pallas_call(
        _body,
        out_shape=jax.ShapeDtypeStruct((SQ, D), jnp.float32),
        in_specs=[pl.BlockSpec(memory_space=pltpu.VMEM)] * 5,
        out_specs=pl.BlockSpec(memory_space=pltpu.VMEM),
        scratch_shapes=[
            pltpu.VMEM((SQ, D), jnp.float32),
            pltpu.VMEM((SQ, D), jnp.float32),
            pltpu.VMEM((N_DEV, CH, D), jnp.float32),
            pltpu.VMEM((N_DEV - 1, CH, D), jnp.float32),
            pltpu.VMEM((N_DEV - 1, CH, D), jnp.float32),
            pltpu.VMEM((N_DEV, CH, D), jnp.float32),
            pltpu.SemaphoreType.DMA((N_STEPS,)),
            pltpu.SemaphoreType.DMA((N_STEPS,)),
            pltpu.SemaphoreType.REGULAR,
        ],
        compiler_params=pltpu.CompilerParams(
            collective_id=0, vmem_limit_bytes=64 * 1024 * 1024),
    )(x[0], Wq, Wo, Kh, Vh)
    return out[None]
